# baseline (speedup 1.0000x reference)
import functools

import jax
import jax.numpy as jnp
from jax.experimental import pallas as pl

TOP_K = 100
_LN_EPS = 1e-5
_B = 128
_BB = 16


def _layernorm_rows(x, g, b):
    m = jnp.mean(x, axis=-1, keepdims=True)
    v = jnp.mean((x - m) ** 2, axis=-1, keepdims=True)
    return (x - m) * jax.lax.rsqrt(v + _LN_EPS) * g + b


def _row_kernel(ft_ref, w_ref, bc_ref, ws_ref, bs_ref, g_ref, b_ref,
                row_ref, ftout_ref):
    ft = ft_ref[:TOP_K, :]
    x = jnp.dot(ft, w_ref[...], preferred_element_type=jnp.float32)
    x = x + bc_ref[0, :]
    row = _layernorm_rows(x, g_ref[...], b_ref[...])
    srow = _layernorm_rows(ws_ref[...] + bs_ref[...], g_ref[...], b_ref[...])
    row_ref[...] = row + srow
    ftout_ref[...] = ft


def _bcast_kernel(row_ref, ft_ref, out_ref, ftout_ref):
    out_ref[...] = jnp.broadcast_to(row_ref[...][None], (_BB, TOP_K, row_ref.shape[-1]))
    ftout_ref[...] = jnp.broadcast_to(ft_ref[...][None], (_BB, TOP_K, ft_ref.shape[-1]))


def kernel(list_clip_image_feat, clip_concepts_feat, fasttext_table,
           W_concept, b_concept, W_score, b_score, ln_g, ln_b):
    del list_clip_image_feat, clip_concepts_feat
    K, D_FT = fasttext_table.shape
    H = W_concept.shape[1]
    rows_pad = ((TOP_K + 7) // 8) * 8

    bc = b_concept.reshape(1, H)
    bs = b_score.reshape(1, H)
    g = ln_g.reshape(1, H)
    b = ln_b.reshape(1, H)

    row, ft_head = pl.pallas_call(
        _row_kernel,
        grid=(1,),
        in_specs=[
            pl.BlockSpec((rows_pad, D_FT), lambda i: (0, 0)),
            pl.BlockSpec((D_FT, H), lambda i: (0, 0)),
            pl.BlockSpec((1, H), lambda i: (0, 0)),
            pl.BlockSpec((1, H), lambda i: (0, 0)),
            pl.BlockSpec((1, H), lambda i: (0, 0)),
            pl.BlockSpec((1, H), lambda i: (0, 0)),
            pl.BlockSpec((1, H), lambda i: (0, 0)),
        ],
        out_specs=[
            pl.BlockSpec((TOP_K, H), lambda i: (0, 0)),
            pl.BlockSpec((TOP_K, D_FT), lambda i: (0, 0)),
        ],
        out_shape=[
            jax.ShapeDtypeStruct((TOP_K, H), jnp.float32),
            jax.ShapeDtypeStruct((TOP_K, D_FT), jnp.float32),
        ],
    )(fasttext_table, W_concept, bc, W_score.reshape(1, H), bs, g, b)

    out, concepts_ft = pl.pallas_call(
        _bcast_kernel,
        grid=(_B // _BB,),
        in_specs=[
            pl.BlockSpec((TOP_K, H), lambda i: (0, 0)),
            pl.BlockSpec((TOP_K, D_FT), lambda i: (0, 0)),
        ],
        out_specs=[
            pl.BlockSpec((_BB, TOP_K, H), lambda i: (i, 0, 0)),
            pl.BlockSpec((_BB, TOP_K, D_FT), lambda i: (i, 0, 0)),
        ],
        out_shape=[
            jax.ShapeDtypeStruct((_B, TOP_K, H), jnp.float32),
            jax.ShapeDtypeStruct((_B, TOP_K, D_FT), jnp.float32),
        ],
    )(row, ft_head)
    return (out, concepts_ft)


# speedup vs baseline: 2.5239x; 2.5239x over previous
import functools

import jax
import jax.numpy as jnp
from jax.experimental import pallas as pl

TOP_K = 100
_LN_EPS = 1e-5
_B = 128
_BB = 16


def _layernorm_rows(x, g, b):
    m = jnp.mean(x, axis=-1, keepdims=True)
    v = jnp.mean((x - m) ** 2, axis=-1, keepdims=True)
    return (x - m) * jax.lax.rsqrt(v + _LN_EPS) * g + b


def _row_kernel(ft_ref, w_ref, bc_ref, ws_ref, bs_ref, g_ref, b_ref,
                row_ref, ftout_ref):
    ft = ft_ref[:TOP_K, :]
    x = jnp.dot(ft, w_ref[...], preferred_element_type=jnp.float32)
    x = x + bc_ref[0, :]
    row = _layernorm_rows(x, g_ref[...], b_ref[...])
    srow = _layernorm_rows(ws_ref[...] + bs_ref[...], g_ref[...], b_ref[...])
    row_ref[...] = row + srow
    ftout_ref[...] = ft


def _bcast_kernel(row_ref, ft_ref, out_ref, ftout_ref):
    out_ref[...] = jnp.broadcast_to(row_ref[...][None], (_BB, TOP_K, row_ref.shape[-1]))
    ftout_ref[...] = jnp.broadcast_to(ft_ref[...][None], (_BB, TOP_K, ft_ref.shape[-1]))


def kernel(list_clip_image_feat, clip_concepts_feat, fasttext_table,
           W_concept, b_concept, W_score, b_score, ln_g, ln_b):
    del list_clip_image_feat, clip_concepts_feat
    K, D_FT = fasttext_table.shape
    H = W_concept.shape[1]
    rows_pad = ((TOP_K + 7) // 8) * 8
    table_head = jax.lax.slice(fasttext_table, (0, 0), (rows_pad, D_FT))

    bc = b_concept.reshape(1, H)
    bs = b_score.reshape(1, H)
    g = ln_g.reshape(1, H)
    b = ln_b.reshape(1, H)

    row, ft_head = pl.pallas_call(
        _row_kernel,
        grid=(1,),
        in_specs=[
            pl.BlockSpec((rows_pad, D_FT), lambda i: (0, 0)),
            pl.BlockSpec((D_FT, H), lambda i: (0, 0)),
            pl.BlockSpec((1, H), lambda i: (0, 0)),
            pl.BlockSpec((1, H), lambda i: (0, 0)),
            pl.BlockSpec((1, H), lambda i: (0, 0)),
            pl.BlockSpec((1, H), lambda i: (0, 0)),
            pl.BlockSpec((1, H), lambda i: (0, 0)),
        ],
        out_specs=[
            pl.BlockSpec((TOP_K, H), lambda i: (0, 0)),
            pl.BlockSpec((TOP_K, D_FT), lambda i: (0, 0)),
        ],
        out_shape=[
            jax.ShapeDtypeStruct((TOP_K, H), jnp.float32),
            jax.ShapeDtypeStruct((TOP_K, D_FT), jnp.float32),
        ],
    )(table_head, W_concept, bc, W_score.reshape(1, H), bs, g, b)

    out, concepts_ft = pl.pallas_call(
        _bcast_kernel,
        grid=(_B // _BB,),
        in_specs=[
            pl.BlockSpec((TOP_K, H), lambda i: (0, 0)),
            pl.BlockSpec((TOP_K, D_FT), lambda i: (0, 0)),
        ],
        out_specs=[
            pl.BlockSpec((_BB, TOP_K, H), lambda i: (i, 0, 0)),
            pl.BlockSpec((_BB, TOP_K, D_FT), lambda i: (i, 0, 0)),
        ],
        out_shape=[
            jax.ShapeDtypeStruct((_B, TOP_K, H), jnp.float32),
            jax.ShapeDtypeStruct((_B, TOP_K, D_FT), jnp.float32),
        ],
    )(row, ft_head)
    return (out, concepts_ft)


# speedup vs baseline: 7.0652x; 2.7994x over previous
import jax
import jax.numpy as jnp
from jax.experimental import pallas as pl

TOP_K = 100
_LN_EPS = 1e-5
_B = 128
_KB = 8
_ROWS_PAD = 104


def _layernorm_rows(x, g, b):
    m = jnp.mean(x, axis=-1, keepdims=True)
    v = jnp.mean((x - m) ** 2, axis=-1, keepdims=True)
    return (x - m) * jax.lax.rsqrt(v + _LN_EPS) * g + b


def _row_kernel(ft_ref, w_ref, bc_ref, ws_ref, bs_ref, g_ref, b_ref,
                row_ref, ftout_ref):
    ft = ft_ref[...]
    x = jnp.dot(ft, w_ref[...], preferred_element_type=jnp.float32)
    x = x + bc_ref[0, :]
    row = _layernorm_rows(x, g_ref[...], b_ref[...])
    srow = _layernorm_rows(ws_ref[...] + bs_ref[...], g_ref[...], b_ref[...])
    row_ref[...] = row + srow
    ftout_ref[...] = ft


def _bcast_kernel(row_ref, ft_ref, out_ref, ftout_ref):
    out_ref[...] = jnp.broadcast_to(row_ref[...][:, None, :],
                                    (_KB, _B, row_ref.shape[-1]))
    ftout_ref[...] = jnp.broadcast_to(ft_ref[...][:, :, None],
                                      (_KB, ft_ref.shape[-1], _B))


def kernel(list_clip_image_feat, clip_concepts_feat, fasttext_table,
           W_concept, b_concept, W_score, b_score, ln_g, ln_b):
    del list_clip_image_feat, clip_concepts_feat
    K, D_FT = fasttext_table.shape
    H = W_concept.shape[1]
    table_head = jax.lax.slice(fasttext_table, (0, 0), (_ROWS_PAD, D_FT))

    bc = b_concept.reshape(1, H)
    bs = b_score.reshape(1, H)
    g = ln_g.reshape(1, H)
    b = ln_b.reshape(1, H)

    row, ft_head = pl.pallas_call(
        _row_kernel,
        grid=(1,),
        in_specs=[
            pl.BlockSpec((_ROWS_PAD, D_FT), lambda i: (0, 0)),
            pl.BlockSpec((D_FT, H), lambda i: (0, 0)),
            pl.BlockSpec((1, H), lambda i: (0, 0)),
            pl.BlockSpec((1, H), lambda i: (0, 0)),
            pl.BlockSpec((1, H), lambda i: (0, 0)),
            pl.BlockSpec((1, H), lambda i: (0, 0)),
            pl.BlockSpec((1, H), lambda i: (0, 0)),
        ],
        out_specs=[
            pl.BlockSpec((_ROWS_PAD, H), lambda i: (0, 0)),
            pl.BlockSpec((_ROWS_PAD, D_FT), lambda i: (0, 0)),
        ],
        out_shape=[
            jax.ShapeDtypeStruct((_ROWS_PAD, H), jnp.float32),
            jax.ShapeDtypeStruct((_ROWS_PAD, D_FT), jnp.float32),
        ],
    )(table_head, W_concept, bc, W_score.reshape(1, H), bs, g, b)

    out_kbh, ft_kdb = pl.pallas_call(
        _bcast_kernel,
        grid=(_ROWS_PAD // _KB,),
        in_specs=[
            pl.BlockSpec((_KB, H), lambda i: (i, 0)),
            pl.BlockSpec((_KB, D_FT), lambda i: (i, 0)),
        ],
        out_specs=[
            pl.BlockSpec((_KB, _B, H), lambda i: (i, 0, 0)),
            pl.BlockSpec((_KB, D_FT, _B), lambda i: (i, 0, 0)),
        ],
        out_shape=[
            jax.ShapeDtypeStruct((TOP_K, _B, H), jnp.float32),
            jax.ShapeDtypeStruct((TOP_K, D_FT, _B), jnp.float32),
        ],
    )(row, ft_head)
    out = jnp.transpose(out_kbh, (1, 0, 2))
    concepts_ft = jnp.transpose(ft_kdb, (2, 0, 1))
    return (out, concepts_ft)


# speedup vs baseline: 8.6598x; 1.2257x over previous
import jax
import jax.numpy as jnp
from jax.experimental import pallas as pl

TOP_K = 100
_LN_EPS = 1e-5
_B = 128
_KB = 8
_ROWS_PAD = 104


def _layernorm_rows(x, g, b):
    m = jnp.mean(x, axis=-1, keepdims=True)
    v = jnp.mean((x - m) ** 2, axis=-1, keepdims=True)
    return (x - m) * jax.lax.rsqrt(v + _LN_EPS) * g + b


def _row_kernel(ft_ref, w_ref, bc_ref, ws_ref, bs_ref, g_ref, b_ref,
                row_ref, ftout_ref):
    ft = ft_ref[...]
    x = jnp.dot(ft, w_ref[...], preferred_element_type=jnp.float32)
    x = x + bc_ref[...]
    row = _layernorm_rows(x, g_ref[...], b_ref[...])
    srow = _layernorm_rows(ws_ref[...] + bs_ref[...], g_ref[...], b_ref[...])
    row_ref[...] = row + srow
    ftout_ref[...] = ft


def _bcast_kernel(row_ref, ft_ref, out_ref, ftout_ref):
    out_ref[...] = jnp.broadcast_to(row_ref[...][:, None, :],
                                    (_KB, _B, row_ref.shape[-1]))
    ftout_ref[...] = jnp.broadcast_to(ft_ref[...][:, :, None],
                                      (_KB, ft_ref.shape[-1], _B))


def kernel(list_clip_image_feat, clip_concepts_feat, fasttext_table,
           W_concept, b_concept, W_score, b_score, ln_g, ln_b):
    del list_clip_image_feat, clip_concepts_feat
    K, D_FT = fasttext_table.shape
    H = W_concept.shape[1]
    table_head = jax.lax.slice(fasttext_table, (0, 0), (_ROWS_PAD, D_FT))

    row, ft_head = pl.pallas_call(
        _row_kernel,
        grid=(1,),
        in_specs=[
            pl.BlockSpec((_ROWS_PAD, D_FT), lambda i: (0, 0)),
            pl.BlockSpec((D_FT, H), lambda i: (0, 0)),
            pl.BlockSpec((H,), lambda i: (0,)),
            pl.BlockSpec((1, H), lambda i: (0, 0)),
            pl.BlockSpec((H,), lambda i: (0,)),
            pl.BlockSpec((H,), lambda i: (0,)),
            pl.BlockSpec((H,), lambda i: (0,)),
        ],
        out_specs=[
            pl.BlockSpec((_ROWS_PAD, H), lambda i: (0, 0)),
            pl.BlockSpec((_ROWS_PAD, D_FT), lambda i: (0, 0)),
        ],
        out_shape=[
            jax.ShapeDtypeStruct((_ROWS_PAD, H), jnp.float32),
            jax.ShapeDtypeStruct((_ROWS_PAD, D_FT), jnp.float32),
        ],
    )(table_head, W_concept, b_concept, W_score, b_score, ln_g, ln_b)

    out_kbh, ft_kdb = pl.pallas_call(
        _bcast_kernel,
        grid=(_ROWS_PAD // _KB,),
        in_specs=[
            pl.BlockSpec((_KB, H), lambda i: (i, 0)),
            pl.BlockSpec((_KB, D_FT), lambda i: (i, 0)),
        ],
        out_specs=[
            pl.BlockSpec((_KB, _B, H), lambda i: (i, 0, 0)),
            pl.BlockSpec((_KB, D_FT, _B), lambda i: (i, 0, 0)),
        ],
        out_shape=[
            jax.ShapeDtypeStruct((TOP_K, _B, H), jnp.float32),
            jax.ShapeDtypeStruct((TOP_K, D_FT, _B), jnp.float32),
        ],
    )(row, ft_head)
    out = jnp.transpose(out_kbh, (1, 0, 2))
    concepts_ft = jnp.transpose(ft_kdb, (2, 0, 1))
    return (out, concepts_ft)


# speedup vs baseline: 9.3882x; 1.0841x over previous
import jax
import jax.numpy as jnp
from jax.experimental import pallas as pl
from jax.experimental.pallas import tpu as pltpu

TOP_K = 100
_LN_EPS = 1e-5
_B = 128
_KB = 8
_ROWS_PAD = 104


def _layernorm_rows(x, g, b):
    m = jnp.mean(x, axis=-1, keepdims=True)
    v = jnp.mean((x - m) ** 2, axis=-1, keepdims=True)
    return (x - m) * jax.lax.rsqrt(v + _LN_EPS) * g + b


def _svoce_kernel(ft_full_ref, ft_blk_ref, w_ref, bc_ref, ws_ref, bs_ref,
                  g_ref, b_ref, out_ref, ftout_ref, row_vmem):
    i = pl.program_id(0)

    @pl.when(i == 0)
    def _compute_rows():
        ft = ft_full_ref[...]
        x = jnp.dot(ft, w_ref[...], preferred_element_type=jnp.float32)
        x = x + bc_ref[...]
        row = _layernorm_rows(x, g_ref[...], b_ref[...])
        srow = _layernorm_rows(ws_ref[...] + bs_ref[...], g_ref[...], b_ref[...])
        row_vmem[...] = row + srow

    rows = row_vmem[pl.ds(i * _KB, _KB), :]
    out_ref[...] = jnp.broadcast_to(rows[:, None, :], (_KB, _B, rows.shape[-1]))
    ftout_ref[...] = jnp.broadcast_to(ft_blk_ref[...][:, :, None],
                                      (_KB, ft_blk_ref.shape[-1], _B))


def kernel(list_clip_image_feat, clip_concepts_feat, fasttext_table,
           W_concept, b_concept, W_score, b_score, ln_g, ln_b):
    del list_clip_image_feat, clip_concepts_feat
    K, D_FT = fasttext_table.shape
    H = W_concept.shape[1]
    table_head = jax.lax.slice(fasttext_table, (0, 0), (_ROWS_PAD, D_FT))

    out_kbh, ft_kdb = pl.pallas_call(
        _svoce_kernel,
        grid=(_ROWS_PAD // _KB,),
        in_specs=[
            pl.BlockSpec((_ROWS_PAD, D_FT), lambda i: (0, 0)),
            pl.BlockSpec((_KB, D_FT), lambda i: (i, 0)),
            pl.BlockSpec((D_FT, H), lambda i: (0, 0)),
            pl.BlockSpec((H,), lambda i: (0,)),
            pl.BlockSpec((1, H), lambda i: (0, 0)),
            pl.BlockSpec((H,), lambda i: (0,)),
            pl.BlockSpec((H,), lambda i: (0,)),
            pl.BlockSpec((H,), lambda i: (0,)),
        ],
        out_specs=[
            pl.BlockSpec((_KB, _B, H), lambda i: (i, 0, 0)),
            pl.BlockSpec((_KB, D_FT, _B), lambda i: (i, 0, 0)),
        ],
        out_shape=[
            jax.ShapeDtypeStruct((TOP_K, _B, H), jnp.float32),
            jax.ShapeDtypeStruct((TOP_K, D_FT, _B), jnp.float32),
        ],
        scratch_shapes=[pltpu.VMEM((_ROWS_PAD, H), jnp.float32)],
    )(table_head, table_head, W_concept, b_concept, W_score, b_score,
      ln_g, ln_b)
    out = jnp.transpose(out_kbh, (1, 0, 2))
    concepts_ft = jnp.transpose(ft_kdb, (2, 0, 1))
    return (out, concepts_ft)
